# baseline (speedup 1.0000x reference)
import jax
import jax.numpy as jnp
from jax.experimental import pallas as pl
from jax.experimental.pallas import tpu as pltpu
from jax.experimental.pallas import tpu_sc as plsc

TOPK = 8
NEG_INF = float("-inf")

_DN_T = (((1,), (1,)), ((), ()))


def _proj_body(x_ref, wq_ref, wk_ref, wv_ref, bq_ref, bk_ref, bv_ref,
               q_ref, k_ref, v_ref):
    xb = x_ref[0].astype(jnp.bfloat16)
    for w_ref, b_ref, o_ref, widen in ((wq_ref, bq_ref, q_ref, False),
                                       (wk_ref, bk_ref, k_ref, False),
                                       (wv_ref, bv_ref, v_ref, True)):
        acc = jax.lax.dot_general(xb, w_ref[...], _DN_T,
                                  preferred_element_type=jnp.float32)
        r = (acc + b_ref[...]).astype(jnp.bfloat16)
        o_ref[0] = r.astype(jnp.float32) if widen else r


def _topk_body(q_ref, k_ref, w_ref, i_ref):
    S = k_ref.shape[1]
    scores = jax.lax.dot_general(q_ref[0], k_ref[0], _DN_T,
                                 preferred_element_type=jnp.float32)
    col = jax.lax.broadcasted_iota(jnp.int32, scores.shape, 1)
    m = jnp.max(scores, axis=1, keepdims=True)
    total = m
    ms = [m]
    idxs = [jnp.min(jnp.where(scores == m, col, S), axis=1, keepdims=True)]
    for _ in range(TOPK - 1):
        m = jnp.max(jnp.where(scores < m, scores, NEG_INF), axis=1,
                    keepdims=True)
        total = total + m
        ms.append(m)
        idxs.append(jnp.min(jnp.where(scores == m, col, S), axis=1,
                            keepdims=True))
    w = jnp.concatenate(ms, axis=1) / (total + 1e-10)
    w_ref[0] = w.astype(jnp.bfloat16).astype(jnp.float32)
    i_ref[0] = jnp.concatenate(idxs, axis=1) + pl.program_id(0) * S


def _make_sc_combine(N, E, QB, per_sub):
    n_items = per_sub // QB

    def sc_combine(v_hbm, i_hbm, w_hbm, o_hbm, idx_v, w_v, g0, g1, o0, o1,
                   sg0, sg1, so0, so1, s_init):
        sid = (jax.lax.axis_index("core") * 16
               + jax.lax.axis_index("subcore"))
        base = sid * (per_sub * TOPK)
        pltpu.make_async_copy(i_hbm.at[:, pl.ds(base, per_sub * TOPK)],
                              idx_v, s_init).start()
        pltpu.make_async_copy(i_hbm.at[:, pl.ds(base, per_sub * TOPK)],
                              idx_v, s_init).wait()
        pltpu.make_async_copy(w_hbm.at[:, pl.ds(base, per_sub * TOPK)],
                              w_v, s_init).start()
        pltpu.make_async_copy(w_hbm.at[:, pl.ds(base, per_sub * TOPK)],
                              w_v, s_init).wait()

        nidx = QB * TOPK

        def g_copy(kk, gbuf, sem):
            return pltpu.make_async_copy(
                v_hbm.at[idx_v.at[0, pl.ds(kk * nidx, nidx)]], gbuf, sem)

        def o_copy(kk, obuf, sem):
            row0 = sid * per_sub + kk * QB
            return pltpu.make_async_copy(obuf, o_hbm.at[pl.ds(row0, QB)], sem)

        g_copy(0, g0, sg0).start()
        g_copy(1, g1, sg1).start()

        def run_item(kk, gbuf, obuf, sgem, soem):
            g_copy(kk, gbuf, sgem).wait()
            wvec = w_v[0, pl.ds(kk * nidx, nidx)]
            for q in range(QB):
                wv = [jnp.full((16,), wvec[q * TOPK + i], jnp.float32)
                      for i in range(TOPK)]

                @pl.loop(0, E, step=16)
                def _(c):
                    s = pl.ds(c, 16)
                    acc = gbuf[q * TOPK, s] * wv[0]
                    for i in range(1, TOPK):
                        acc = acc + gbuf[q * TOPK + i, s] * wv[i]
                    obuf[q, s] = acc
            o_copy(kk, obuf, soem).start()

        @pl.loop(0, n_items // 2)
        def _(it):
            kk0 = it * 2
            kk1 = kk0 + 1

            @pl.when(it > 0)
            def _():
                o_copy(kk0 - 2, o0, so0).wait()
                o_copy(kk1 - 2, o1, so1).wait()

            run_item(kk0, g0, o0, sg0, so0)

            @pl.when(kk0 + 2 < n_items)
            def _():
                g_copy(kk0 + 2, g0, sg0).start()

            run_item(kk1, g1, o1, sg1, so1)

            @pl.when(kk1 + 2 < n_items)
            def _():
                g_copy(kk1 + 2, g1, sg1).start()

        o_copy(n_items - 2, o0, so0).wait()
        o_copy(n_items - 1, o1, so1).wait()

    return sc_combine


def kernel(x, Wq, bq, Wk, bk, Wv, bv):
    B, S, E = x.shape
    MBLK = min(512, S)
    SBLK = min(512, S)
    nm = S // MBLK
    N = B * S

    wq = Wq.astype(jnp.bfloat16)
    wk = Wk.astype(jnp.bfloat16)
    wv = Wv.astype(jnp.bfloat16)
    b2 = lambda b: b.reshape(1, E)

    w_spec = pl.BlockSpec((E, E), lambda i: (0, 0))
    b_spec = pl.BlockSpec((1, E), lambda i: (0, 0))
    row_spec = pl.BlockSpec((1, MBLK, E), lambda i: (i // nm, i % nm, 0))
    bf_sd = jax.ShapeDtypeStruct((B, S, E), jnp.bfloat16)
    f32_sd = jax.ShapeDtypeStruct((B, S, E), jnp.float32)

    q, kk, vv = pl.pallas_call(
        _proj_body,
        grid=(B * nm,),
        in_specs=[row_spec, w_spec, w_spec, w_spec, b_spec, b_spec, b_spec],
        out_specs=[row_spec, row_spec, row_spec],
        out_shape=[bf_sd, bf_sd, f32_sd],
    )(x, wq, wk, wv, b2(bq), b2(bk), b2(bv))

    w8, gi = pl.pallas_call(
        _topk_body,
        grid=(B, S // SBLK),
        in_specs=[
            pl.BlockSpec((1, SBLK, E), lambda b, i: (b, i, 0)),
            pl.BlockSpec((1, S, E), lambda b, i: (b, 0, 0)),
        ],
        out_specs=[
            pl.BlockSpec((1, SBLK, TOPK), lambda b, i: (b, i, 0)),
            pl.BlockSpec((1, SBLK, TOPK), lambda b, i: (b, i, 0)),
        ],
        out_shape=[jax.ShapeDtypeStruct((B, S, TOPK), jnp.float32),
                   jax.ShapeDtypeStruct((B, S, TOPK), jnp.int32)],
    )(q, kk)

    QB = 2
    per_sub = N // 32
    mesh = plsc.VectorSubcoreMesh(core_axis_name="core",
                                  subcore_axis_name="subcore")
    sc = pl.kernel(
        _make_sc_combine(N, E, QB, per_sub),
        out_type=jax.ShapeDtypeStruct((N, E), jnp.float32),
        mesh=mesh,
        scratch_types=[
            pltpu.VMEM((1, per_sub * TOPK), jnp.int32),
            pltpu.VMEM((1, per_sub * TOPK), jnp.float32),
            pltpu.VMEM((QB * TOPK, E), jnp.float32),
            pltpu.VMEM((QB * TOPK, E), jnp.float32),
            pltpu.VMEM((QB, E), jnp.float32),
            pltpu.VMEM((QB, E), jnp.float32),
            pltpu.SemaphoreType.DMA,
            pltpu.SemaphoreType.DMA,
            pltpu.SemaphoreType.DMA,
            pltpu.SemaphoreType.DMA,
            pltpu.SemaphoreType.DMA,
        ],
    )
    out = sc(vv.reshape(N, E), gi.reshape(1, N * TOPK),
             w8.reshape(1, N * TOPK))
    return out.reshape(B, S, E)


# speedup vs baseline: 1.9376x; 1.9376x over previous
import jax
import jax.numpy as jnp
from jax.experimental import pallas as pl
from jax.experimental.pallas import tpu as pltpu

TOPK = 8
NEG_INF = float("-inf")

_DN_T = (((1,), (1,)), ((), ()))
_DN = (((1,), (0,)), ((), ()))


def _proj_body(x_ref, wq_ref, wk_ref, wv_ref, bq_ref, bk_ref, bv_ref,
               q_ref, k_ref, v_ref):
    xb = x_ref[0].astype(jnp.bfloat16)
    for w_ref, b_ref, o_ref in ((wq_ref, bq_ref, q_ref),
                                (wk_ref, bk_ref, k_ref),
                                (wv_ref, bv_ref, v_ref)):
        acc = jax.lax.dot_general(xb, w_ref[...], _DN_T,
                                  preferred_element_type=jnp.float32)
        o_ref[0] = (acc + b_ref[...]).astype(jnp.bfloat16)


def _make_attn_body(nsb):
    def _attn_body(q_ref, k_ref, v_ref, o_ref, w0, w1):
        i = pl.program_id(1)
        odd = i % 2 == 1

        def _combine(buf):
            o_ref[0] = jax.lax.dot_general(
                buf[...], v_ref[0], _DN,
                preferred_element_type=jnp.float32)

        @pl.when(jnp.logical_and(i > 0, odd))
        def _():
            _combine(w0)

        @pl.when(jnp.logical_and(i > 0, jnp.logical_not(odd)))
        def _():
            _combine(w1)

        @pl.when(i < nsb)
        def _():
            scores = jax.lax.dot_general(q_ref[0], k_ref[0], _DN_T,
                                         preferred_element_type=jnp.float32)
            m = jnp.max(scores, axis=1, keepdims=True)
            total = m
            for _ in range(TOPK - 1):
                m = jnp.max(jnp.where(scores < m, scores, NEG_INF), axis=1,
                            keepdims=True)
                total = total + m
            w = jnp.where(scores >= m, scores, 0.0) / (total + 1e-10)
            wbf = w.astype(jnp.bfloat16)

            @pl.when(jnp.logical_not(odd))
            def _():
                w0[...] = wbf

            @pl.when(odd)
            def _():
                w1[...] = wbf

    return _attn_body


def kernel(x, Wq, bq, Wk, bk, Wv, bv):
    B, S, E = x.shape
    MBLK = min(512, S)
    SBLK = min(512, S)
    nm = S // MBLK

    wq = Wq.astype(jnp.bfloat16)
    wk = Wk.astype(jnp.bfloat16)
    wv = Wv.astype(jnp.bfloat16)
    b2 = lambda b: b.reshape(1, E)

    w_spec = pl.BlockSpec((E, E), lambda i: (0, 0))
    b_spec = pl.BlockSpec((1, E), lambda i: (0, 0))
    row_spec = pl.BlockSpec((1, MBLK, E), lambda i: (i // nm, i % nm, 0))
    out_sd = jax.ShapeDtypeStruct((B, S, E), jnp.bfloat16)

    q, kk, vv = pl.pallas_call(
        _proj_body,
        grid=(B * nm,),
        in_specs=[row_spec, w_spec, w_spec, w_spec, b_spec, b_spec, b_spec],
        out_specs=[row_spec, row_spec, row_spec],
        out_shape=[out_sd, out_sd, out_sd],
    )(x, wq, wk, wv, b2(bq), b2(bk), b2(bv))

    nsb = S // SBLK
    out = pl.pallas_call(
        _make_attn_body(nsb),
        grid=(B, nsb + 1),
        in_specs=[
            pl.BlockSpec((1, SBLK, E),
                         lambda b, i: (b, jnp.minimum(i, nsb - 1), 0)),
            pl.BlockSpec((1, S, E), lambda b, i: (b, 0, 0)),
            pl.BlockSpec((1, S, E), lambda b, i: (b, 0, 0)),
        ],
        out_specs=pl.BlockSpec((1, SBLK, E),
                               lambda b, i: (b, jnp.maximum(i - 1, 0), 0)),
        out_shape=jax.ShapeDtypeStruct((B, S, E), jnp.float32),
        scratch_shapes=[pltpu.VMEM((SBLK, S), jnp.bfloat16),
                        pltpu.VMEM((SBLK, S), jnp.bfloat16)],
    )(q, kk, vv)
    return out


# speedup vs baseline: 2.0137x; 1.0393x over previous
import jax
import jax.numpy as jnp
from jax.experimental import pallas as pl

TOPK = 8
NEG_INF = float("-inf")

_DN_T = (((1,), (1,)), ((), ()))
_DN = (((1,), (0,)), ((), ()))


def _proj_body(x_ref, wq_ref, wk_ref, wv_ref, bq_ref, bk_ref, bv_ref,
               q_ref, k_ref, v_ref):
    xb = x_ref[0].astype(jnp.bfloat16)
    for w_ref, b_ref, o_ref in ((wq_ref, bq_ref, q_ref),
                                (wk_ref, bk_ref, k_ref),
                                (wv_ref, bv_ref, v_ref)):
        acc = jax.lax.dot_general(xb, w_ref[...], _DN_T,
                                  preferred_element_type=jnp.float32)
        o_ref[0] = (acc + b_ref[...]).astype(jnp.bfloat16)


def _attn_body(q_ref, k_ref, v_ref, o_ref):
    q = q_ref[0]
    k = k_ref[0]
    scores = jax.lax.dot_general(q, k, _DN_T,
                                 preferred_element_type=jnp.float32)
    m = jnp.max(scores, axis=1, keepdims=True)
    total = m
    for _ in range(TOPK - 1):
        m = jnp.max(jnp.where(scores < m, scores, NEG_INF), axis=1,
                    keepdims=True)
        total = total + m
    w = jnp.where(scores >= m, scores, 0.0) / (total + 1e-10)
    o_ref[0] = jax.lax.dot_general(w.astype(jnp.bfloat16), v_ref[0], _DN,
                                   preferred_element_type=jnp.float32)


def kernel(x, Wq, bq, Wk, bk, Wv, bv):
    B, S, E = x.shape
    MBLK = min(512, S)
    SBLK = min(512, S)
    nm = S // MBLK

    wq = Wq.astype(jnp.bfloat16)
    wk = Wk.astype(jnp.bfloat16)
    wv = Wv.astype(jnp.bfloat16)
    b2 = lambda b: b.reshape(1, E)

    w_spec = pl.BlockSpec((E, E), lambda i: (0, 0))
    b_spec = pl.BlockSpec((1, E), lambda i: (0, 0))
    row_spec = pl.BlockSpec((1, MBLK, E), lambda i: (i // nm, i % nm, 0))
    out_sd = jax.ShapeDtypeStruct((B, S, E), jnp.bfloat16)

    q, kk, vv = pl.pallas_call(
        _proj_body,
        grid=(B * nm,),
        in_specs=[row_spec, w_spec, w_spec, w_spec, b_spec, b_spec, b_spec],
        out_specs=[row_spec, row_spec, row_spec],
        out_shape=[out_sd, out_sd, out_sd],
    )(x, wq, wk, wv, b2(bq), b2(bk), b2(bv))

    out = pl.pallas_call(
        _attn_body,
        grid=(B, S // SBLK),
        in_specs=[
            pl.BlockSpec((1, SBLK, E), lambda b, i: (b, i, 0)),
            pl.BlockSpec((1, S, E), lambda b, i: (b, 0, 0)),
            pl.BlockSpec((1, S, E), lambda b, i: (b, 0, 0)),
        ],
        out_specs=pl.BlockSpec((1, SBLK, E), lambda b, i: (b, i, 0)),
        out_shape=jax.ShapeDtypeStruct((B, S, E), jnp.float32),
    )(q, kk, vv)
    return out
